# baseline (speedup 1.0000x reference)
import functools

import jax
import jax.numpy as jnp
from jax import lax
from jax.experimental import pallas as pl
from jax.experimental.pallas import tpu as pltpu

_KTOP = 8
_CONF = 0.95

_M = 256
_D = 131072
_B = 2
_NAUG = 16
_C = 19
_HW = 128 * 128

_CH1 = 8192
_NJ1 = _D // _CH1
_CH2 = 8192
_NJ2 = _D // _CH2
_BLK = 4096
_NP = _HW // _BLK


def _k1_body(q_ref, t_ref, dist_ref, acc_ref):
    j = pl.program_id(0)
    q = q_ref[...]
    t = t_ref[...]
    d0 = jnp.sum((q - t[0:1, :]) ** 2, axis=1)
    d1 = jnp.sum((q - t[1:2, :]) ** 2, axis=1)
    part = jnp.concatenate([d0.reshape(1, _M), d1.reshape(1, _M)], axis=0)

    @pl.when(j == 0)
    def _():
        acc_ref[...] = part

    @pl.when(j > 0)
    def _():
        acc_ref[...] = acc_ref[...] + part

    @pl.when(j == _NJ1 - 1)
    def _():
        dist_ref[...] = jnp.sqrt(acc_ref[...])


def _k2a_body(d_ref, midx_ref, close_ref):
    d = d_ref[...]
    mn = jnp.min(d, axis=1, keepdims=True)
    iot = lax.broadcasted_iota(jnp.int32, (_B, _M), 1)
    mi = jnp.min(jnp.where(d == mn, iot, _M), axis=1, keepdims=True)
    midx_ref[...] = mi
    close_ref[...] = mn


def _k2b_body(midx_ref, q_ref, a_ref, c_ref, w_ref, nv_ref, acc_ref):
    b = pl.program_id(0)
    j = pl.program_id(1)
    qrow = q_ref[0, 0, 0, :].reshape(1, _CH2)
    a = a_ref[0]
    part = jnp.sum((a - qrow) ** 2, axis=1).reshape(1, _NAUG)

    @pl.when(j == 0)
    def _():
        acc_ref[...] = part

    @pl.when(j > 0)
    def _():
        acc_ref[...] = acc_ref[...] + part

    @pl.when(j == _NJ2 - 1)
    def _():
        d = jnp.sqrt(acc_ref[...])
        cd = c_ref[b, 0]
        maskv = d <= cd
        nvalid = jnp.sum(maskv.astype(jnp.float32))
        dm = jnp.where(maskv, d, jnp.inf)
        niot = lax.broadcasted_iota(jnp.int32, (1, _NAUG), 1)
        rank = jnp.zeros((1, _NAUG), jnp.int32)
        for s in range(1, _NAUG):
            r = pltpu.roll(dm, s, 1)
            miot = (niot - s) % _NAUG
            cond = (r < dm) | ((r == dm) & (miot < niot))
            rank = rank + cond.astype(jnp.int32)
        sel = maskv & (rank < _KTOP)
        w_ref[0, 0, :] = sel.astype(jnp.float32)[0]
        nv_ref[0, 0, :] = jnp.full((_NAUG,), nvalid, jnp.float32)


def _k3_body(w_ref, nv_ref, alog_ref, tlog_ref, plab_ref, out_ref, acc_ref):
    b = pl.program_id(0)
    n = pl.program_id(2)
    x = alog_ref[0, 0]
    m = jnp.max(x, axis=0, keepdims=True)
    e = jnp.exp(x - m)
    s = jnp.sum(e, axis=0, keepdims=True)
    probs = e / s
    wn = w_ref[b, n]
    contrib = wn * probs

    @pl.when(n == 0)
    def _():
        acc_ref[...] = contrib

    @pl.when(n > 0)
    def _():
        acc_ref[...] = acc_ref[...] + contrib

    @pl.when(n == _NAUG - 1)
    def _():
        t = tlog_ref[0]
        tm = jnp.max(t, axis=0, keepdims=True)
        te = jnp.exp(t - tm)
        ts = jnp.sum(te, axis=0, keepdims=True)
        mp = jnp.max(te / ts, axis=0)
        pmask = mp < _CONF
        acc = acc_ref[...]
        mx = jnp.max(acc, axis=0, keepdims=True)
        iot = lax.broadcasted_iota(jnp.int32, (_C, _BLK), 0)
        arg = jnp.min(jnp.where(acc == mx, iot, _C), axis=0)
        valid = nv_ref[b] > 0.0
        plab = plab_ref[0, 0]
        out_ref[0, 0, :] = jnp.where(pmask & valid, arg, plab)


def kernel(source_queue, tgt_feat, tgt_logits, auged_feat, auged_logits,
           pseudo_label):
    queue_flat = source_queue.reshape(_M, _D)
    tgt_flat = tgt_feat.reshape(_B, _D)

    dist = pl.pallas_call(
        _k1_body,
        grid=(_NJ1,),
        in_specs=[
            pl.BlockSpec((_M, _CH1), lambda j: (0, j)),
            pl.BlockSpec((_B, _CH1), lambda j: (0, j)),
        ],
        out_specs=pl.BlockSpec((_B, _M), lambda j: (0, 0)),
        out_shape=jax.ShapeDtypeStruct((_B, _M), jnp.float32),
        scratch_shapes=[pltpu.VMEM((_B, _M), jnp.float32)],
    )(queue_flat, tgt_flat)

    midx, closest = pl.pallas_call(
        _k2a_body,
        out_shape=(
            jax.ShapeDtypeStruct((_B, 1), jnp.int32),
            jax.ShapeDtypeStruct((_B, 1), jnp.float32),
        ),
    )(dist)

    queue4 = queue_flat.reshape(_M, _NJ2, 1, _CH2)
    aug_flat = auged_feat.reshape(_B, _NAUG, _D)

    w_out, nv_out = pl.pallas_call(
        _k2b_body,
        grid_spec=pltpu.PrefetchScalarGridSpec(
            num_scalar_prefetch=1,
            grid=(_B, _NJ2),
            in_specs=[
                pl.BlockSpec((1, 1, 1, _CH2),
                             lambda b, j, midx: (midx[b, 0], j, 0, 0)),
                pl.BlockSpec((1, _NAUG, _CH2), lambda b, j, midx: (b, 0, j)),
                pl.BlockSpec((_B, 1), lambda b, j, midx: (0, 0)),
            ],
            out_specs=(
                pl.BlockSpec((1, 1, _NAUG), lambda b, j, midx: (b, 0, 0)),
                pl.BlockSpec((1, 1, _NAUG), lambda b, j, midx: (b, 0, 0)),
            ),
            scratch_shapes=[pltpu.VMEM((1, _NAUG), jnp.float32)],
        ),
        out_shape=(
            jax.ShapeDtypeStruct((_B, 1, _NAUG), jnp.float32),
            jax.ShapeDtypeStruct((_B, 1, _NAUG), jnp.float32),
        ),
    )(midx, queue4, aug_flat, closest)

    w = w_out[:, 0, :]
    nv = nv_out[:, 0, 0]

    alog = auged_logits.reshape(_B, _NAUG, _C, _HW)
    tlog = tgt_logits.reshape(_B, _C, _HW)
    plab3 = pseudo_label.reshape(_B, 1, _HW)

    out = pl.pallas_call(
        _k3_body,
        grid=(_B, _NP, _NAUG),
        in_specs=[
            pl.BlockSpec(memory_space=pltpu.SMEM),
            pl.BlockSpec(memory_space=pltpu.SMEM),
            pl.BlockSpec((1, 1, _C, _BLK), lambda b, p, n: (b, n, 0, p)),
            pl.BlockSpec((1, _C, _BLK), lambda b, p, n: (b, 0, p)),
            pl.BlockSpec((1, 1, _BLK), lambda b, p, n: (b, 0, p)),
        ],
        out_specs=pl.BlockSpec((1, 1, _BLK), lambda b, p, n: (b, 0, p)),
        out_shape=jax.ShapeDtypeStruct((_B, 1, _HW), jnp.int32),
        scratch_shapes=[pltpu.VMEM((_C, _BLK), jnp.float32)],
    )(w, nv, alog, tlog, plab3)

    return out.reshape(_B, 128, 128)
